# baseline (speedup 1.0000x reference)
import functools

import jax
import jax.numpy as jnp
from jax import lax
from jax.experimental import pallas as pl
from jax.experimental.pallas import tpu as pltpu
from jax.experimental.pallas import tpu_sc as plsc

N = 10000
D = 128
E = 320000

NC = 2
NS = 16
CHUNK = 128
NPAD = 10112
RPT = NPAD // NS
DBLK = 8
EPT = 10240
NCHUNK = EPT // CHUNK
EP = NC * NS * EPT
DEGW = 16

@functools.lru_cache(maxsize=None)
def _sc_mesh():
    return plsc.VectorSubcoreMesh(core_axis_name="c", subcore_axis_name="s",
                                  num_cores=NC, num_subcores=NS)


def _fill_const(ref, nrows, ncols, val):
    v = jnp.full((16,), val, jnp.float32)

    def body(r, _):
        for j in range(ncols // 16):
            ref[r, pl.ds(j * 16, 16)] = v
        return 0

    lax.fori_loop(0, nrows, body, 0, unroll=False)


def _zero_shared_slice(buf, nrows, ncols, shared, base):
    _fill_const(buf, nrows, ncols, 0.0)
    full, rem = RPT // nrows, RPT % nrows
    for k in range(full):
        pltpu.sync_copy(buf, shared.at[pl.ds(base + k * nrows, nrows)])
    if rem:
        pltpu.sync_copy(buf.at[pl.ds(0, rem)],
                        shared.at[pl.ds(base + full * nrows, rem)])


def _deg_body(dst_hbm, deg_out, acc_sh, dst_v, ones_v):
    c = lax.axis_index("c")
    s = lax.axis_index("s")

    _zero_shared_slice(ones_v, CHUNK, D, acc_sh, s * RPT)
    _fill_const(ones_v, CHUNK, D, 1.0)
    plsc.subcore_barrier()

    def outer(g, _):
        pltpu.sync_copy(dst_hbm.at[c, s, pl.ds(g * DBLK, DBLK), :], dst_v)
        for jj in range(DBLK):
            pltpu.sync_copy(ones_v, acc_sh.at[dst_v.at[jj]], add=True)
        return 0

    lax.fori_loop(0, NCHUNK // DBLK, outer, 0, unroll=False)
    plsc.subcore_barrier()
    pltpu.sync_copy(acc_sh.at[pl.ds(s * RPT, RPT)],
                    deg_out.at[c, pl.ds(s * RPT, RPT)])


def _deg_call(dstr):
    return pl.kernel(
        _deg_body,
        out_type=jax.ShapeDtypeStruct((NC, NPAD, D), jnp.float32),
        mesh=_sc_mesh(),
        scratch_types=[
            pltpu.VMEM_SHARED((NPAD, D), jnp.float32),
            pltpu.VMEM((DBLK, CHUNK), jnp.int32),
            pltpu.VMEM((CHUNK, D), jnp.float32),
        ],
    )(dstr)


K0C = 160
TOTCH = EP // CHUNK


def _prop_body(s_hbm, src_hbm, dst_hbm, acc_out, acc_sh, src_v, dst_v, rows0,
               rows1, gsem, ssem):
    c = lax.axis_index("c")
    s = lax.axis_index("s")

    _zero_shared_slice(rows0, CHUNK, D, acc_sh, s * RPT)
    plsc.subcore_barrier()

    bufs = (rows0, rows1)
    nout = jnp.where(c == 0, K0C // DBLK, 0)
    cbase = s * K0C

    def gather(jj, buf):
        return pltpu.async_copy(
            s_hbm.at[src_v.at[pl.ds(jj * CHUNK, CHUNK)]], buf, gsem)

    def outer(g, _):
        ch0 = cbase + g * DBLK
        pltpu.sync_copy(src_hbm.at[pl.ds(ch0 * CHUNK, DBLK * CHUNK)], src_v)
        pltpu.sync_copy(dst_hbm.at[pl.ds(ch0, DBLK), :], dst_v)
        gd = [gather(0, bufs[0]), gather(1, bufs[1])]
        sd = [None] * DBLK
        for jj in range(DBLK):
            if jj >= 1 and jj + 1 < DBLK:
                sd[jj - 1].wait()
                gd.append(gather(jj + 1, bufs[(jj + 1) % 2]))
            gd[jj].wait()
            sd[jj] = pltpu.async_copy(bufs[jj % 2], acc_sh.at[dst_v.at[jj]],
                                      ssem, add=True)
        sd[DBLK - 2].wait()
        sd[DBLK - 1].wait()
        return 0

    lax.fori_loop(0, nout, outer, 0, unroll=False)
    plsc.subcore_barrier()

    @pl.when(c == 0)
    def _():
        pltpu.sync_copy(acc_sh.at[pl.ds(s * RPT, RPT)],
                        acc_out.at[pl.ds(s * RPT, RPT)])


def _prop_call(s, srcf, dstc):
    return pl.kernel(
        _prop_body,
        out_type=jax.ShapeDtypeStruct((NPAD, D), jnp.float32),
        mesh=_sc_mesh(),
        scratch_types=[
            pltpu.VMEM_SHARED((NPAD, D), jnp.float32),
            pltpu.VMEM((DBLK * CHUNK,), jnp.int32),
            pltpu.VMEM((DBLK, CHUNK), jnp.int32),
            pltpu.VMEM((CHUNK, D), jnp.float32),
            pltpu.VMEM((CHUNK, D), jnp.float32),
            pltpu.SemaphoreType.DMA,
            pltpu.SemaphoreType.DMA,
        ],
    )(s, srcf, dstc)


RB = 400
GRID = N // RB


def _mm0_body(x_ref, w_ref, d0_ref, d1_ref, s_ref, dinv_ref):
    dinv = lax.rsqrt(d0_ref[...] + d1_ref[...] + 1.0)
    h = jnp.dot(x_ref[...], w_ref[...], preferred_element_type=jnp.float32)
    s_ref[...] = h * dinv
    dinv_ref[...] = dinv


def _mm0(x, w, d0, d1):
    return pl.pallas_call(
        _mm0_body,
        grid=(GRID,),
        in_specs=[
            pl.BlockSpec((RB, D), lambda i: (i, 0)),
            pl.BlockSpec((D, D), lambda i: (0, 0)),
            pl.BlockSpec((RB, 1), lambda i: (i, 0)),
            pl.BlockSpec((RB, 1), lambda i: (i, 0)),
        ],
        out_specs=[
            pl.BlockSpec((RB, D), lambda i: (i, 0)),
            pl.BlockSpec((RB, 1), lambda i: (i, 0)),
        ],
        out_shape=[
            jax.ShapeDtypeStruct((N, D), jnp.float32),
            jax.ShapeDtypeStruct((N, 1), jnp.float32),
        ],
    )(x, w, d0, d1)


def _post_body(a0_ref, s_ref, dinv_ref, b_ref, o_ref, st_ref, *, relu):
    h = (a0_ref[...] + s_ref[...]) * dinv_ref[...] + b_ref[...]
    if relu:
        h = jnp.maximum(h, 0.0)
    o_ref[...] = h
    cs = jnp.sum(h, axis=0, keepdims=True)
    cq = jnp.sum(h * h, axis=0, keepdims=True)
    st = jnp.concatenate([cs, cq, jnp.zeros((6, D), jnp.float32)], axis=0)

    @pl.when(pl.program_id(0) == 0)
    def _():
        st_ref[...] = jnp.zeros_like(st_ref)

    st_ref[...] += st


def _post(a0, s, dinv, b, relu):
    return pl.pallas_call(
        functools.partial(_post_body, relu=relu),
        grid=(GRID,),
        in_specs=[
            pl.BlockSpec((RB, D), lambda i: (i, 0)),
            pl.BlockSpec((RB, D), lambda i: (i, 0)),
            pl.BlockSpec((RB, 1), lambda i: (i, 0)),
            pl.BlockSpec((1, D), lambda i: (0, 0)),
        ],
        out_specs=[
            pl.BlockSpec((RB, D), lambda i: (i, 0)),
            pl.BlockSpec((8, D), lambda i: (0, 0)),
        ],
        out_shape=[
            jax.ShapeDtypeStruct((N, D), jnp.float32),
            jax.ShapeDtypeStruct((8, D), jnp.float32),
        ],
    )(a0, s, dinv, b)


def _mmbn_body(o_ref, st_ref, w_ref, dinv_ref, out_ref):
    mu = st_ref[0:1, :] * (1.0 / N)
    var = st_ref[1:2, :] * (1.0 / N) - mu * mu
    cbn = lax.rsqrt(var + 1e-5)
    xn = (o_ref[...] - mu) * cbn
    out_ref[...] = jnp.dot(xn, w_ref[...],
                           preferred_element_type=jnp.float32) * dinv_ref[...]


def _mmbn(o, st, w, dinv):
    return pl.pallas_call(
        _mmbn_body,
        grid=(GRID,),
        in_specs=[
            pl.BlockSpec((RB, D), lambda i: (i, 0)),
            pl.BlockSpec((8, D), lambda i: (0, 0)),
            pl.BlockSpec((D, D), lambda i: (0, 0)),
            pl.BlockSpec((RB, 1), lambda i: (i, 0)),
        ],
        out_specs=pl.BlockSpec((RB, D), lambda i: (i, 0)),
        out_shape=jax.ShapeDtypeStruct((N, D), jnp.float32),
    )(o, st, w, dinv)


def _final_body(h_ref, st_ref, wc_ref, bc_ref, out_ref, acc_ref):
    i = pl.program_id(0)

    @pl.when(i == 0)
    def _():
        acc_ref[...] = jnp.zeros_like(acc_ref)

    mu = st_ref[0:1, :] * (1.0 / N)
    var = st_ref[1:2, :] * (1.0 / N) - mu * mu
    cbn = lax.rsqrt(var + 1e-5)
    xn = (h_ref[...] - mu) * cbn
    acc_ref[...] += jnp.concatenate(
        [jnp.sum(xn, axis=0, keepdims=True), jnp.zeros((7, D), jnp.float32)],
        axis=0)

    @pl.when(i == pl.num_programs(0) - 1)
    def _():
        pooled = acc_ref[0:1, :] * (1.0 / N)
        logits = jnp.dot(pooled, wc_ref[...],
                         preferred_element_type=jnp.float32) + bc_ref[...]
        m = jnp.max(logits, axis=1, keepdims=True)
        e = jnp.exp(logits - m)
        out_ref[...] = e / jnp.sum(e, axis=1, keepdims=True)


def _final(h, st, wc, bc):
    c = wc.shape[1]
    return pl.pallas_call(
        _final_body,
        grid=(GRID,),
        in_specs=[
            pl.BlockSpec((RB, D), lambda i: (i, 0)),
            pl.BlockSpec((8, D), lambda i: (0, 0)),
            pl.BlockSpec((D, c), lambda i: (0, 0)),
            pl.BlockSpec((1, c), lambda i: (0, 0)),
        ],
        out_specs=pl.BlockSpec((1, c), lambda i: (0, 0)),
        out_shape=jax.ShapeDtypeStruct((1, c), jnp.float32),
        scratch_shapes=[pltpu.VMEM((8, D), jnp.float32)],
    )(h, st, wc, bc)


def kernel(x, edge_index, W0, b0, W1, b1, W2, b2, Wc, bc):
    src = edge_index[0]
    dst = edge_index[1]
    pad = EP - E
    src_p = jnp.concatenate([src, jnp.zeros((pad,), jnp.int32)])
    dst_p = jnp.concatenate([dst, jnp.full((pad,), N, jnp.int32)])
    srcf = src_p
    dstc = dst_p.reshape(TOTCH, CHUNK)
    dstr = dst_p.reshape(NC, NS, NCHUNK, CHUNK)
    deg = _deg_call(dstr)
    d0 = deg[0, :N, 0:1]
    d1 = deg[1, :N, 0:1]

    s0, dinv = _mm0(x, W0, d0, d1)
    acc = _prop_call(s0, srcf, dstc)
    o0, st0 = _post(acc[:N], s0, dinv, b0.reshape(1, -1), True)

    s1 = _mmbn(o0, st0, W1, dinv)
    acc = _prop_call(s1, srcf, dstc)
    o1, st1 = _post(acc[:N], s1, dinv, b1.reshape(1, -1), True)

    s2 = _mmbn(o1, st1, W2, dinv)
    acc = _prop_call(s2, srcf, dstc)
    h3, st2 = _post(acc[:N], s2, dinv, b2.reshape(1, -1), False)

    return _final(h3, st2, Wc, bc.reshape(1, -1))


# speedup vs baseline: 1.2484x; 1.2484x over previous
import functools

import jax
import jax.numpy as jnp
from jax import lax
from jax.experimental import pallas as pl
from jax.experimental.pallas import tpu as pltpu
from jax.experimental.pallas import tpu_sc as plsc

N = 10000
D = 128
E = 320000

NC = 2
NS = 16
CHUNK = 128
NPAD = 10112
RPT = NPAD // NS
DBLK = 8
EPT = 10240
NCHUNK = EPT // CHUNK
EP = NC * NS * EPT
DEGW = 16

@functools.lru_cache(maxsize=None)
def _sc_mesh():
    return plsc.VectorSubcoreMesh(core_axis_name="c", subcore_axis_name="s",
                                  num_cores=NC, num_subcores=NS)


def _fill_const(ref, nrows, ncols, val):
    v = jnp.full((16,), val, jnp.float32)

    def body(r, _):
        for j in range(ncols // 16):
            ref[r, pl.ds(j * 16, 16)] = v
        return 0

    lax.fori_loop(0, nrows, body, 0, unroll=False)


def _zero_shared_slice(buf, nrows, ncols, shared, base):
    _fill_const(buf, nrows, ncols, 0.0)
    full, rem = RPT // nrows, RPT % nrows
    for k in range(full):
        pltpu.sync_copy(buf, shared.at[pl.ds(base + k * nrows, nrows)])
    if rem:
        pltpu.sync_copy(buf.at[pl.ds(0, rem)],
                        shared.at[pl.ds(base + full * nrows, rem)])


def _deg_body(dst_hbm, deg_out, acc_sh, dst_v, ones_v):
    c = lax.axis_index("c")
    s = lax.axis_index("s")

    _zero_shared_slice(ones_v, CHUNK, D, acc_sh, s * RPT)
    _fill_const(ones_v, CHUNK, D, 1.0)
    plsc.subcore_barrier()

    def outer(g, _):
        pltpu.sync_copy(dst_hbm.at[c, s, pl.ds(g * DBLK, DBLK), :], dst_v)
        for jj in range(DBLK):
            pltpu.sync_copy(ones_v, acc_sh.at[dst_v.at[jj]], add=True)
        return 0

    lax.fori_loop(0, NCHUNK // DBLK, outer, 0, unroll=False)
    plsc.subcore_barrier()
    pltpu.sync_copy(acc_sh.at[pl.ds(s * RPT, RPT)],
                    deg_out.at[c, pl.ds(s * RPT, RPT)])


def _deg_call(dstr):
    return pl.kernel(
        _deg_body,
        out_type=jax.ShapeDtypeStruct((NC, NPAD, D), jnp.float32),
        mesh=_sc_mesh(),
        scratch_types=[
            pltpu.VMEM_SHARED((NPAD, D), jnp.float32),
            pltpu.VMEM((DBLK, CHUNK), jnp.int32),
            pltpu.VMEM((CHUNK, D), jnp.float32),
        ],
    )(dstr)


K0C = 152
K1C = NCHUNK * NC - K0C
CB1 = NS * K0C
TOTCH = EP // CHUNK


def _prop_body(s_hbm, src_hbm, dst_hbm, acc_out, acc_sh, src_v, dst_v, rows0,
               rows1, gsem, ssem):
    c = lax.axis_index("c")
    s = lax.axis_index("s")

    _zero_shared_slice(rows0, CHUNK, D, acc_sh, s * RPT)
    plsc.subcore_barrier()

    bufs = (rows0, rows1)
    nout = jnp.where(c == 0, K0C // DBLK, K1C // DBLK)
    cbase = jnp.where(c == 0, s * K0C, CB1 + s * K1C)

    def gather(jj, buf):
        return pltpu.async_copy(
            s_hbm.at[src_v.at[pl.ds(jj * CHUNK, CHUNK)]], buf, gsem)

    def outer(g, _):
        ch0 = cbase + g * DBLK
        pltpu.sync_copy(src_hbm.at[pl.ds(ch0 * CHUNK, DBLK * CHUNK)], src_v)
        pltpu.sync_copy(dst_hbm.at[pl.ds(ch0, DBLK), :], dst_v)
        gd = [gather(0, bufs[0]), gather(1, bufs[1])]
        sd = [None] * DBLK
        for jj in range(DBLK):
            if jj >= 1 and jj + 1 < DBLK:
                sd[jj - 1].wait()
                gd.append(gather(jj + 1, bufs[(jj + 1) % 2]))
            gd[jj].wait()
            sd[jj] = pltpu.async_copy(bufs[jj % 2], acc_sh.at[dst_v.at[jj]],
                                      ssem, add=True)
        sd[DBLK - 2].wait()
        sd[DBLK - 1].wait()
        return 0

    lax.fori_loop(0, nout, outer, 0, unroll=False)
    plsc.subcore_barrier()
    pltpu.sync_copy(acc_sh.at[pl.ds(s * RPT, RPT)],
                    acc_out.at[c, pl.ds(s * RPT, RPT)])


def _prop_call(s, srcf, dstc):
    return pl.kernel(
        _prop_body,
        out_type=jax.ShapeDtypeStruct((NC, NPAD, D), jnp.float32),
        mesh=_sc_mesh(),
        scratch_types=[
            pltpu.VMEM_SHARED((NPAD, D), jnp.float32),
            pltpu.VMEM((DBLK * CHUNK,), jnp.int32),
            pltpu.VMEM((DBLK, CHUNK), jnp.int32),
            pltpu.VMEM((CHUNK, D), jnp.float32),
            pltpu.VMEM((CHUNK, D), jnp.float32),
            pltpu.SemaphoreType.DMA,
            pltpu.SemaphoreType.DMA,
        ],
    )(s, srcf, dstc)


RB = 400
GRID = N // RB


def _mm0_body(x_ref, w_ref, d0_ref, d1_ref, s_ref, dinv_ref):
    dinv = lax.rsqrt(d0_ref[...] + d1_ref[...] + 1.0)
    h = jnp.dot(x_ref[...], w_ref[...], preferred_element_type=jnp.float32)
    s_ref[...] = h * dinv
    dinv_ref[...] = dinv


def _mm0(x, w, d0, d1):
    return pl.pallas_call(
        _mm0_body,
        grid=(GRID,),
        in_specs=[
            pl.BlockSpec((RB, D), lambda i: (i, 0)),
            pl.BlockSpec((D, D), lambda i: (0, 0)),
            pl.BlockSpec((RB, 1), lambda i: (i, 0)),
            pl.BlockSpec((RB, 1), lambda i: (i, 0)),
        ],
        out_specs=[
            pl.BlockSpec((RB, D), lambda i: (i, 0)),
            pl.BlockSpec((RB, 1), lambda i: (i, 0)),
        ],
        out_shape=[
            jax.ShapeDtypeStruct((N, D), jnp.float32),
            jax.ShapeDtypeStruct((N, 1), jnp.float32),
        ],
    )(x, w, d0, d1)


def _post_body(a0_ref, a1_ref, s_ref, dinv_ref, b_ref, o_ref, st_ref, *, relu):
    h = (a0_ref[...] + a1_ref[...] + s_ref[...]) * dinv_ref[...] + b_ref[...]
    if relu:
        h = jnp.maximum(h, 0.0)
    o_ref[...] = h
    cs = jnp.sum(h, axis=0, keepdims=True)
    cq = jnp.sum(h * h, axis=0, keepdims=True)
    st = jnp.concatenate([cs, cq, jnp.zeros((6, D), jnp.float32)], axis=0)

    @pl.when(pl.program_id(0) == 0)
    def _():
        st_ref[...] = jnp.zeros_like(st_ref)

    st_ref[...] += st


def _post(a0, a1, s, dinv, b, relu):
    return pl.pallas_call(
        functools.partial(_post_body, relu=relu),
        grid=(GRID,),
        in_specs=[
            pl.BlockSpec((RB, D), lambda i: (i, 0)),
            pl.BlockSpec((RB, D), lambda i: (i, 0)),
            pl.BlockSpec((RB, D), lambda i: (i, 0)),
            pl.BlockSpec((RB, 1), lambda i: (i, 0)),
            pl.BlockSpec((1, D), lambda i: (0, 0)),
        ],
        out_specs=[
            pl.BlockSpec((RB, D), lambda i: (i, 0)),
            pl.BlockSpec((8, D), lambda i: (0, 0)),
        ],
        out_shape=[
            jax.ShapeDtypeStruct((N, D), jnp.float32),
            jax.ShapeDtypeStruct((8, D), jnp.float32),
        ],
    )(a0, a1, s, dinv, b)


def _mmbn_body(o_ref, st_ref, w_ref, dinv_ref, out_ref):
    mu = st_ref[0:1, :] * (1.0 / N)
    var = st_ref[1:2, :] * (1.0 / N) - mu * mu
    cbn = lax.rsqrt(var + 1e-5)
    xn = (o_ref[...] - mu) * cbn
    out_ref[...] = jnp.dot(xn, w_ref[...],
                           preferred_element_type=jnp.float32) * dinv_ref[...]


def _mmbn(o, st, w, dinv):
    return pl.pallas_call(
        _mmbn_body,
        grid=(GRID,),
        in_specs=[
            pl.BlockSpec((RB, D), lambda i: (i, 0)),
            pl.BlockSpec((8, D), lambda i: (0, 0)),
            pl.BlockSpec((D, D), lambda i: (0, 0)),
            pl.BlockSpec((RB, 1), lambda i: (i, 0)),
        ],
        out_specs=pl.BlockSpec((RB, D), lambda i: (i, 0)),
        out_shape=jax.ShapeDtypeStruct((N, D), jnp.float32),
    )(o, st, w, dinv)


def _final_body(h_ref, st_ref, wc_ref, bc_ref, out_ref, acc_ref):
    i = pl.program_id(0)

    @pl.when(i == 0)
    def _():
        acc_ref[...] = jnp.zeros_like(acc_ref)

    mu = st_ref[0:1, :] * (1.0 / N)
    var = st_ref[1:2, :] * (1.0 / N) - mu * mu
    cbn = lax.rsqrt(var + 1e-5)
    xn = (h_ref[...] - mu) * cbn
    acc_ref[...] += jnp.concatenate(
        [jnp.sum(xn, axis=0, keepdims=True), jnp.zeros((7, D), jnp.float32)],
        axis=0)

    @pl.when(i == pl.num_programs(0) - 1)
    def _():
        pooled = acc_ref[0:1, :] * (1.0 / N)
        logits = jnp.dot(pooled, wc_ref[...],
                         preferred_element_type=jnp.float32) + bc_ref[...]
        m = jnp.max(logits, axis=1, keepdims=True)
        e = jnp.exp(logits - m)
        out_ref[...] = e / jnp.sum(e, axis=1, keepdims=True)


def _final(h, st, wc, bc):
    c = wc.shape[1]
    return pl.pallas_call(
        _final_body,
        grid=(GRID,),
        in_specs=[
            pl.BlockSpec((RB, D), lambda i: (i, 0)),
            pl.BlockSpec((8, D), lambda i: (0, 0)),
            pl.BlockSpec((D, c), lambda i: (0, 0)),
            pl.BlockSpec((1, c), lambda i: (0, 0)),
        ],
        out_specs=pl.BlockSpec((1, c), lambda i: (0, 0)),
        out_shape=jax.ShapeDtypeStruct((1, c), jnp.float32),
        scratch_shapes=[pltpu.VMEM((8, D), jnp.float32)],
    )(h, st, wc, bc)


def kernel(x, edge_index, W0, b0, W1, b1, W2, b2, Wc, bc):
    src = edge_index[0]
    dst = edge_index[1]
    pad = EP - E
    src_p = jnp.concatenate([src, jnp.zeros((pad,), jnp.int32)])
    dst_p = jnp.concatenate([dst, jnp.full((pad,), N, jnp.int32)])
    srcf = src_p
    dstc = dst_p.reshape(TOTCH, CHUNK)
    dstr = dst_p.reshape(NC, NS, NCHUNK, CHUNK)
    deg = _deg_call(dstr)
    d0 = deg[0, :N, 0:1]
    d1 = deg[1, :N, 0:1]

    s0, dinv = _mm0(x, W0, d0, d1)
    acc = _prop_call(s0, srcf, dstc)
    o0, st0 = _post(acc[0, :N], acc[1, :N], s0, dinv, b0.reshape(1, -1), True)

    s1 = _mmbn(o0, st0, W1, dinv)
    acc = _prop_call(s1, srcf, dstc)
    o1, st1 = _post(acc[0, :N], acc[1, :N], s1, dinv, b1.reshape(1, -1), True)

    s2 = _mmbn(o1, st1, W2, dinv)
    acc = _prop_call(s2, srcf, dstc)
    h3, st2 = _post(acc[0, :N], acc[1, :N], s2, dinv, b2.reshape(1, -1), False)

    return _final(h3, st2, Wc, bc.reshape(1, -1))


# speedup vs baseline: 1.3080x; 1.0478x over previous
import functools

import jax
import jax.numpy as jnp
from jax import lax
from jax.experimental import pallas as pl
from jax.experimental.pallas import tpu as pltpu
from jax.experimental.pallas import tpu_sc as plsc

N = 10000
D = 128
E = 320000

NC = 2
NS = 16
CHUNK = 128
NPAD = 10112
RPT = NPAD // NS
DBLK = 8
EPT = 10240
NCHUNK = EPT // CHUNK
EP = NC * NS * EPT
DEGW = 16

@functools.lru_cache(maxsize=None)
def _sc_mesh():
    return plsc.VectorSubcoreMesh(core_axis_name="c", subcore_axis_name="s",
                                  num_cores=NC, num_subcores=NS)


def _fill_const(ref, nrows, ncols, val):
    v = jnp.full((16,), val, jnp.float32)

    def body(r, _):
        for j in range(ncols // 16):
            ref[r, pl.ds(j * 16, 16)] = v
        return 0

    lax.fori_loop(0, nrows, body, 0, unroll=False)


def _zero_shared_slice(buf, nrows, ncols, shared, base):
    _fill_const(buf, nrows, ncols, 0.0)
    full, rem = RPT // nrows, RPT % nrows
    for k in range(full):
        pltpu.sync_copy(buf, shared.at[pl.ds(base + k * nrows, nrows)])
    if rem:
        pltpu.sync_copy(buf.at[pl.ds(0, rem)],
                        shared.at[pl.ds(base + full * nrows, rem)])


def _deg_body(dst_hbm, deg_out, acc_sh, dst_v, ones_v):
    c = lax.axis_index("c")
    s = lax.axis_index("s")

    _zero_shared_slice(ones_v, CHUNK, D, acc_sh, s * RPT)
    _fill_const(ones_v, CHUNK, D, 1.0)
    plsc.subcore_barrier()

    def outer(g, _):
        pltpu.sync_copy(dst_hbm.at[c, s, pl.ds(g * DBLK, DBLK), :], dst_v)
        for jj in range(DBLK):
            pltpu.sync_copy(ones_v, acc_sh.at[dst_v.at[jj]], add=True)
        return 0

    lax.fori_loop(0, NCHUNK // DBLK, outer, 0, unroll=False)
    plsc.subcore_barrier()
    pltpu.sync_copy(acc_sh.at[pl.ds(s * RPT, RPT)],
                    deg_out.at[c, pl.ds(s * RPT, RPT)])


def _deg_call(dstr):
    return pl.kernel(
        _deg_body,
        out_type=jax.ShapeDtypeStruct((NC, NPAD, D), jnp.float32),
        mesh=_sc_mesh(),
        scratch_types=[
            pltpu.VMEM_SHARED((NPAD, D), jnp.float32),
            pltpu.VMEM((DBLK, CHUNK), jnp.int32),
            pltpu.VMEM((CHUNK, D), jnp.float32),
        ],
    )(dstr)


K0C = 152
K1C = NCHUNK * NC - K0C
CB1 = NS * K0C
TOTCH = EP // CHUNK


def _prop_body(s_hbm, src_hbm, dst_hbm, acc_out, acc_sh, src_v, dst_v, rows0,
               rows1, gsem, ssem):
    c = lax.axis_index("c")
    s = lax.axis_index("s")

    _zero_shared_slice(rows0, CHUNK, D, acc_sh, s * RPT)
    plsc.subcore_barrier()

    bufs = (rows0, rows1)
    nout = jnp.where(c == 0, K0C // DBLK, K1C // DBLK)
    cbase = jnp.where(c == 0, s * K0C, CB1 + s * K1C)

    def gather(jj, buf):
        return pltpu.async_copy(
            s_hbm.at[src_v.at[pl.ds(jj * CHUNK, CHUNK)]], buf, gsem)

    def outer(g, _):
        ch0 = cbase + g * DBLK
        pltpu.sync_copy(src_hbm.at[pl.ds(ch0 * CHUNK, DBLK * CHUNK)], src_v)
        pltpu.sync_copy(dst_hbm.at[pl.ds(ch0, DBLK), :], dst_v)
        gd = [gather(0, bufs[0]), gather(1, bufs[1])]
        sd = [None] * DBLK
        for jj in range(DBLK):
            if jj >= 1 and jj + 1 < DBLK:
                sd[jj - 1].wait()
                gd.append(gather(jj + 1, bufs[(jj + 1) % 2]))
            gd[jj].wait()
            sd[jj] = pltpu.async_copy(bufs[jj % 2], acc_sh.at[dst_v.at[jj]],
                                      ssem, add=True)
        sd[DBLK - 2].wait()
        sd[DBLK - 1].wait()
        return 0

    lax.fori_loop(0, nout, outer, 0, unroll=False)
    plsc.subcore_barrier()
    pltpu.sync_copy(acc_sh.at[pl.ds(s * RPT, RPT)],
                    acc_out.at[c, pl.ds(s * RPT, RPT)])


def _prop_call(s, srcf, dstc):
    return pl.kernel(
        _prop_body,
        out_type=jax.ShapeDtypeStruct((NC, NPAD, D), jnp.float32),
        mesh=_sc_mesh(),
        scratch_types=[
            pltpu.VMEM_SHARED((NPAD, D), jnp.float32),
            pltpu.VMEM((DBLK * CHUNK,), jnp.int32),
            pltpu.VMEM((DBLK, CHUNK), jnp.int32),
            pltpu.VMEM((CHUNK, D), jnp.float32),
            pltpu.VMEM((CHUNK, D), jnp.float32),
            pltpu.SemaphoreType.DMA,
            pltpu.SemaphoreType.DMA,
        ],
    )(s, srcf, dstc)


RB = 2000
GRID = N // RB


def _mm0_body(x_ref, w_ref, d0_ref, d1_ref, s_ref, dinv_ref):
    dinv = lax.rsqrt(d0_ref[...] + d1_ref[...] + 1.0)
    h = jnp.dot(x_ref[...], w_ref[...], preferred_element_type=jnp.float32)
    s_ref[...] = h * dinv
    dinv_ref[...] = dinv


def _mm0(x, w, d0, d1):
    return pl.pallas_call(
        _mm0_body,
        grid=(GRID,),
        in_specs=[
            pl.BlockSpec((RB, D), lambda i: (i, 0)),
            pl.BlockSpec((D, D), lambda i: (0, 0)),
            pl.BlockSpec((RB, 1), lambda i: (i, 0)),
            pl.BlockSpec((RB, 1), lambda i: (i, 0)),
        ],
        out_specs=[
            pl.BlockSpec((RB, D), lambda i: (i, 0)),
            pl.BlockSpec((RB, 1), lambda i: (i, 0)),
        ],
        out_shape=[
            jax.ShapeDtypeStruct((N, D), jnp.float32),
            jax.ShapeDtypeStruct((N, 1), jnp.float32),
        ],
    )(x, w, d0, d1)


def _post_body(a0_ref, a1_ref, s_ref, dinv_ref, b_ref, o_ref, st_ref, *, relu):
    h = (a0_ref[...] + a1_ref[...] + s_ref[...]) * dinv_ref[...] + b_ref[...]
    if relu:
        h = jnp.maximum(h, 0.0)
    o_ref[...] = h
    cs = jnp.sum(h, axis=0, keepdims=True)
    cq = jnp.sum(h * h, axis=0, keepdims=True)
    st = jnp.concatenate([cs, cq, jnp.zeros((6, D), jnp.float32)], axis=0)

    @pl.when(pl.program_id(0) == 0)
    def _():
        st_ref[...] = jnp.zeros_like(st_ref)

    st_ref[...] += st


def _post(a0, a1, s, dinv, b, relu):
    return pl.pallas_call(
        functools.partial(_post_body, relu=relu),
        grid=(GRID,),
        in_specs=[
            pl.BlockSpec((RB, D), lambda i: (i, 0)),
            pl.BlockSpec((RB, D), lambda i: (i, 0)),
            pl.BlockSpec((RB, D), lambda i: (i, 0)),
            pl.BlockSpec((RB, 1), lambda i: (i, 0)),
            pl.BlockSpec((1, D), lambda i: (0, 0)),
        ],
        out_specs=[
            pl.BlockSpec((RB, D), lambda i: (i, 0)),
            pl.BlockSpec((8, D), lambda i: (0, 0)),
        ],
        out_shape=[
            jax.ShapeDtypeStruct((N, D), jnp.float32),
            jax.ShapeDtypeStruct((8, D), jnp.float32),
        ],
    )(a0, a1, s, dinv, b)


def _mmbn_body(o_ref, st_ref, w_ref, dinv_ref, out_ref):
    mu = st_ref[0:1, :] * (1.0 / N)
    var = st_ref[1:2, :] * (1.0 / N) - mu * mu
    cbn = lax.rsqrt(var + 1e-5)
    xn = (o_ref[...] - mu) * cbn
    out_ref[...] = jnp.dot(xn, w_ref[...],
                           preferred_element_type=jnp.float32) * dinv_ref[...]


def _mmbn(o, st, w, dinv):
    return pl.pallas_call(
        _mmbn_body,
        grid=(GRID,),
        in_specs=[
            pl.BlockSpec((RB, D), lambda i: (i, 0)),
            pl.BlockSpec((8, D), lambda i: (0, 0)),
            pl.BlockSpec((D, D), lambda i: (0, 0)),
            pl.BlockSpec((RB, 1), lambda i: (i, 0)),
        ],
        out_specs=pl.BlockSpec((RB, D), lambda i: (i, 0)),
        out_shape=jax.ShapeDtypeStruct((N, D), jnp.float32),
    )(o, st, w, dinv)


def _final_body(h_ref, st_ref, wc_ref, bc_ref, out_ref, acc_ref):
    i = pl.program_id(0)

    @pl.when(i == 0)
    def _():
        acc_ref[...] = jnp.zeros_like(acc_ref)

    mu = st_ref[0:1, :] * (1.0 / N)
    var = st_ref[1:2, :] * (1.0 / N) - mu * mu
    cbn = lax.rsqrt(var + 1e-5)
    xn = (h_ref[...] - mu) * cbn
    acc_ref[...] += jnp.concatenate(
        [jnp.sum(xn, axis=0, keepdims=True), jnp.zeros((7, D), jnp.float32)],
        axis=0)

    @pl.when(i == pl.num_programs(0) - 1)
    def _():
        pooled = acc_ref[0:1, :] * (1.0 / N)
        logits = jnp.dot(pooled, wc_ref[...],
                         preferred_element_type=jnp.float32) + bc_ref[...]
        m = jnp.max(logits, axis=1, keepdims=True)
        e = jnp.exp(logits - m)
        out_ref[...] = e / jnp.sum(e, axis=1, keepdims=True)


def _final(h, st, wc, bc):
    c = wc.shape[1]
    return pl.pallas_call(
        _final_body,
        grid=(GRID,),
        in_specs=[
            pl.BlockSpec((RB, D), lambda i: (i, 0)),
            pl.BlockSpec((8, D), lambda i: (0, 0)),
            pl.BlockSpec((D, c), lambda i: (0, 0)),
            pl.BlockSpec((1, c), lambda i: (0, 0)),
        ],
        out_specs=pl.BlockSpec((1, c), lambda i: (0, 0)),
        out_shape=jax.ShapeDtypeStruct((1, c), jnp.float32),
        scratch_shapes=[pltpu.VMEM((8, D), jnp.float32)],
    )(h, st, wc, bc)


def kernel(x, edge_index, W0, b0, W1, b1, W2, b2, Wc, bc):
    src = edge_index[0]
    dst = edge_index[1]
    pad = EP - E
    src_p = jnp.concatenate([src, jnp.zeros((pad,), jnp.int32)])
    dst_p = jnp.concatenate([dst, jnp.full((pad,), N, jnp.int32)])
    srcf = src_p
    dstc = dst_p.reshape(TOTCH, CHUNK)
    dstr = dst_p.reshape(NC, NS, NCHUNK, CHUNK)
    deg = _deg_call(dstr)
    d0 = deg[0, :N, 0:1]
    d1 = deg[1, :N, 0:1]

    s0, dinv = _mm0(x, W0, d0, d1)
    acc = _prop_call(s0, srcf, dstc)
    o0, st0 = _post(acc[0, :N], acc[1, :N], s0, dinv, b0.reshape(1, -1), True)

    s1 = _mmbn(o0, st0, W1, dinv)
    acc = _prop_call(s1, srcf, dstc)
    o1, st1 = _post(acc[0, :N], acc[1, :N], s1, dinv, b1.reshape(1, -1), True)

    s2 = _mmbn(o1, st1, W2, dinv)
    acc = _prop_call(s2, srcf, dstc)
    h3, st2 = _post(acc[0, :N], acc[1, :N], s2, dinv, b2.reshape(1, -1), False)

    return _final(h3, st2, Wc, bc.reshape(1, -1))
